# baseline (speedup 1.0000x reference)
import jax
import jax.numpy as jnp
from jax.experimental import pallas as pl
from jax.experimental.pallas import tpu as pltpu

NUM_EXPERTS = 64
EMBED_DIM = 2048
N_TOKENS = 16384

BLOCK_M = 1024


def _router_kernel(x_ref, wt_ref, b_ref, o_ref):
    logits = jnp.dot(x_ref[...], wt_ref[...],
                     preferred_element_type=jnp.float32)
    logits = logits + b_ref[...]
    m = jnp.max(logits, axis=-1, keepdims=True)
    e = jnp.exp(logits - m)
    o_ref[...] = e / jnp.sum(e, axis=-1, keepdims=True)


def kernel(x, W, b):
    n_tokens, embed_dim = x.shape
    n_experts = W.shape[0]
    wt = W.T
    b2 = b.reshape(1, n_experts)
    grid = (n_tokens // BLOCK_M,)
    return pl.pallas_call(
        _router_kernel,
        grid=grid,
        in_specs=[
            pl.BlockSpec((BLOCK_M, embed_dim), lambda i: (i, 0)),
            pl.BlockSpec((embed_dim, n_experts), lambda i: (0, 0)),
            pl.BlockSpec((1, n_experts), lambda i: (0, 0)),
        ],
        out_specs=pl.BlockSpec((BLOCK_M, n_experts), lambda i: (i, 0)),
        out_shape=jax.ShapeDtypeStruct((n_tokens, n_experts), jnp.float32),
        compiler_params=pltpu.CompilerParams(
            dimension_semantics=("arbitrary",),
        ),
    )(x, wt, b2)


# speedup vs baseline: 1.0059x; 1.0059x over previous
import jax
import jax.numpy as jnp
from jax.experimental import pallas as pl
from jax.experimental.pallas import tpu as pltpu

NUM_EXPERTS = 64
EMBED_DIM = 2048
N_TOKENS = 16384

BLOCK_M = 1024


def _router_kernel(x_ref, wt_ref, b_ref, o_ref):
    logits = jnp.dot(x_ref[...].astype(jnp.bfloat16), wt_ref[...],
                     preferred_element_type=jnp.float32)
    logits = logits + b_ref[...]
    m = jnp.max(logits, axis=-1, keepdims=True)
    e = jnp.exp(logits - m)
    o_ref[...] = e / jnp.sum(e, axis=-1, keepdims=True)


def kernel(x, W, b):
    n_tokens, embed_dim = x.shape
    n_experts = W.shape[0]
    wt = W.T.astype(jnp.bfloat16)
    b2 = b.reshape(1, n_experts)
    grid = (n_tokens // BLOCK_M,)
    return pl.pallas_call(
        _router_kernel,
        grid=grid,
        in_specs=[
            pl.BlockSpec((BLOCK_M, embed_dim), lambda i: (i, 0)),
            pl.BlockSpec((embed_dim, n_experts), lambda i: (0, 0)),
            pl.BlockSpec((1, n_experts), lambda i: (0, 0)),
        ],
        out_specs=pl.BlockSpec((BLOCK_M, n_experts), lambda i: (i, 0)),
        out_shape=jax.ShapeDtypeStruct((n_tokens, n_experts), jnp.float32),
        compiler_params=pltpu.CompilerParams(
            dimension_semantics=("arbitrary",),
        ),
    )(x, wt, b2)
